# baseline (device time: 46246 ns/iter reference)
import jax
import jax.numpy as jnp
from jax import lax
from jax.experimental import pallas as pl
from jax.experimental.pallas import tpu as pltpu

N_DEV = 4
N_COL_BLOCKS = 4


def kernel(x, w_mat, scale_x, scale_w):
    m_total, k_shard = x.shape
    k_total, n = w_mat.shape
    m_per = m_total // N_DEV
    n_blk = n // N_COL_BLOCKS

    def body(x_ref, w_ref, sx_ref, sw_ref, out_ref,
             comm_ref, wv_ref, xl_ref, acc_ref, epi_ref,
             send_sems, recv_sems, wsems, xsem, osems):
        my = lax.axis_index("i")

        barrier_sem = pltpu.get_barrier_semaphore()
        for o in range(1, N_DEV):
            peer = lax.rem(my + o, N_DEV)
            pl.semaphore_signal(
                barrier_sem, inc=1,
                device_id=(peer,), device_id_type=pl.DeviceIdType.MESH,
            )
        pl.semaphore_wait(barrier_sem, N_DEV - 1)

        def make_send(o):
            j = lax.rem(my + o, N_DEV)
            return pltpu.make_async_remote_copy(
                src_ref=x_ref.at[pl.ds(j * m_per, m_per), :],
                dst_ref=comm_ref.at[my],
                send_sem=send_sems.at[o - 1],
                recv_sem=recv_sems.at[my],
                device_id=(j,),
                device_id_type=pl.DeviceIdType.MESH,
            )

        send_r = make_send(1)
        send_l = make_send(3)
        send_r.start()
        send_l.start()

        xl_copy = pltpu.make_async_copy(
            x_ref.at[pl.ds(my * m_per, m_per), :], xl_ref, xsem)
        xl_copy.start()
        w_copies = []
        for d in range(N_DEV):
            c = pltpu.make_async_copy(
                w_ref.at[pl.ds(d * k_shard, k_shard), :],
                wv_ref.at[d], wsems.at[d])
            c.start()
            w_copies.append(c)

        def chunk_dot(a, b):
            return lax.dot_general(
                a, b, (((1,), (0,)), ((), ())),
                preferred_element_type=jnp.int32,
            )

        xl_copy.wait()
        for c in w_copies:
            c.wait()
        acc_ref[...] = chunk_dot(xl_ref[...], wv_ref[my])

        send_r.wait_send()
        send_l.wait_send()
        send_d = make_send(2)
        send_d.start()

        def recv_from(d):
            return pltpu.make_async_remote_copy(
                src_ref=comm_ref.at[d],
                dst_ref=comm_ref.at[d],
                send_sem=send_sems.at[0],
                recv_sem=recv_sems.at[d],
                device_id=(d,),
                device_id_type=pl.DeviceIdType.MESH,
            )

        for o in (1, 3):
            d = lax.rem(my + N_DEV - o, N_DEV)
            recv_from(d).wait_recv()
            acc_ref[...] += chunk_dot(comm_ref[d], wv_ref[d])

        d2 = lax.rem(my + 2, N_DEV)
        recv_from(d2).wait_recv()
        scale = sx_ref[0] * sw_ref[0]
        out_dmas = []
        for c in range(N_COL_BLOCKS):
            cs = pl.ds(c * n_blk, n_blk)
            acc_ref[:, cs] += chunk_dot(comm_ref[d2], wv_ref[d2, :, cs])
            y = acc_ref[:, cs].astype(jnp.float32) * scale
            epi_ref[:, cs] = y * (1.0 / (1.0 + jnp.exp(-jnp.clip(y, -60.0, 60.0))))
            dma = pltpu.make_async_copy(
                epi_ref.at[:, cs], out_ref.at[:, cs], osems.at[c])
            dma.start()
            out_dmas.append(dma)

        for dma in out_dmas:
            dma.wait()
        send_d.wait_send()

    return pl.pallas_call(
        body,
        out_shape=jax.ShapeDtypeStruct((m_per, n), jnp.float32),
        in_specs=[
            pl.BlockSpec(memory_space=pl.ANY),
            pl.BlockSpec(memory_space=pl.ANY),
            pl.BlockSpec(memory_space=pltpu.SMEM),
            pl.BlockSpec(memory_space=pltpu.SMEM),
        ],
        out_specs=pl.BlockSpec(memory_space=pl.ANY),
        scratch_shapes=[
            pltpu.VMEM((N_DEV, m_per, k_shard), jnp.int8),
            pltpu.VMEM((N_DEV, k_shard, n), jnp.int8),
            pltpu.VMEM((m_per, k_shard), jnp.int8),
            pltpu.VMEM((m_per, n), jnp.int32),
            pltpu.VMEM((m_per, n), jnp.float32),
            pltpu.SemaphoreType.DMA((N_DEV - 1,)),
            pltpu.SemaphoreType.DMA((N_DEV,)),
            pltpu.SemaphoreType.DMA((N_DEV,)),
            pltpu.SemaphoreType.DMA,
            pltpu.SemaphoreType.DMA((N_COL_BLOCKS,)),
        ],
        compiler_params=pltpu.CompilerParams(collective_id=0),
    )(x, w_mat, scale_x, scale_w)
